# baseline (speedup 1.0000x reference)
import functools

import jax
import jax.numpy as jnp
from jax import lax
from jax.experimental import pallas as pl
from jax.experimental.pallas import tpu as pltpu
from jax.experimental.pallas import tpu_sc as plsc

B = 16384
NC, NS = 2, 16
NW = NC * NS
BPT = B // (NW // 2)
GSZ = 16
TFLAT = 10 * 1024
EPS = 1e-5


def _sc_gather_body(tabs_hbm, out_hbm, tv, iv, ov, sem):
  wid = lax.axis_index("s") * NC + lax.axis_index("c")
  half = wid // (NW // 2)
  base = (wid % (NW // 2)) * BPT

  c1 = pltpu.async_copy(tabs_hbm.at[pl.ds(half * TFLAT, TFLAT)], tv, sem)
  c2 = pltpu.async_copy(
      tabs_hbm.at[pl.ds(2 * TFLAT + half * B + base, BPT)], iv, sem)
  c1.wait()
  c2.wait()

  @plsc.parallel_loop(0, BPT // GSZ, unroll=8)
  def _group(g):
    sl = pl.ds(g * GSZ, GSZ)
    pc = plsc.bitcast(iv[sl], jnp.int32)
    for d in range(10):
      ov[d, sl] = plsc.load_gather(tv, [pc + d * 1024])

  zeros = jnp.zeros((GSZ,), jnp.float32)
  for k in range(BPT // GSZ):
    sl = pl.ds(k * GSZ, GSZ)
    for d in range(10, 16):
      ov[d, sl] = zeros

  pltpu.sync_copy(ov, out_hbm.at[pl.ds(half * 16, 16), pl.ds(base, BPT)])


def _sc_gather(tabs):
  mesh = plsc.VectorSubcoreMesh(core_axis_name="c", subcore_axis_name="s")
  fn = functools.partial(
      pl.kernel,
      out_type=jax.ShapeDtypeStruct((32, B), jnp.float32),
      mesh=mesh,
      scratch_types=[
          pltpu.VMEM((TFLAT,), jnp.float32),
          pltpu.VMEM((BPT,), jnp.float32),
          pltpu.VMEM((16, BPT), jnp.float32),
          pltpu.SemaphoreType.DMA,
      ],
      compiler_params=pltpu.CompilerParams(needs_layout_passes=False),
  )(_sc_gather_body)
  return fn(tabs)


def _tc_dense_body(co_hbm, xt_ref, wall_ref, out_ref, co_v, sem):
  pltpu.make_async_copy(co_hbm, co_v, sem).start()
  inv_b = 1.0 / B

  def bn(xp, g, b):
    m = jnp.sum(xp, axis=1, keepdims=True) * inv_b
    d = xp - m
    v = jnp.sum(d * d, axis=1, keepdims=True) * inv_b
    s = g * lax.rsqrt(v + EPS)
    return d * s + b

  pc = jnp.transpose(wall_ref[16:32, 48:80])
  wall = wall_ref[...]
  pltpu.make_async_copy(co_hbm, co_v, sem).wait()
  co = co_v[...]
  xt = xt_ref[...]
  xc = bn(co[0:16], pc[0:16, 0:1], pc[0:16, 1:2])
  xo = bn(co[16:32], pc[0:16, 2:3], pc[0:16, 3:4])
  xn = bn(xt, pc[0:10, 4:5], pc[0:10, 5:6])
  y1 = (jnp.dot(wall[:, 0:16], xc, preferred_element_type=jnp.float32)
        + jnp.dot(wall[:, 16:32], xo, preferred_element_type=jnp.float32)
        + jnp.dot(wall[:, 32:42], xn, preferred_element_type=jnp.float32)
        + pc[:, 6:7])
  h1 = jnp.maximum(y1, 0.0)
  h2 = jnp.maximum(
      jnp.dot(wall[0:16, 48:80], h1, preferred_element_type=jnp.float32)
      + pc[0:16, 7:8], 0.0)
  out_ref[...] = (jnp.dot(wall[0:4, 80:96], h2,
                          preferred_element_type=jnp.float32)
                  + pc[0:4, 8:9])


def _tc_dense(co, xt, wall, interpret=False):
  return pl.pallas_call(
      _tc_dense_body,
      out_shape=jax.ShapeDtypeStruct((4, B), jnp.float32),
      in_specs=[
          pl.BlockSpec(memory_space=pltpu.HBM),
          pl.BlockSpec(memory_space=pltpu.VMEM),
          pl.BlockSpec(memory_space=pltpu.VMEM),
      ],
      scratch_shapes=[
          pltpu.VMEM((32, B), jnp.float32),
          pltpu.SemaphoreType.DMA,
      ],
      interpret=interpret,
  )(co, xt, wall)


def _flat_table(tab):
  return jnp.pad(tab.T, ((0, 0), (0, 24))).reshape(TFLAT)


def _pack_params(bn_gamma, bn_beta, W1, b1, W2, b2, W3, b3):
  z6 = jnp.zeros((6,), jnp.float32)
  z16 = jnp.zeros((16,), jnp.float32)
  z2 = jnp.zeros((2,), jnp.float32)

  def spread10(v6):
    return jnp.concatenate([v6[0:1], z2, v6[1:6], z2])

  prow = jnp.stack([
      jnp.concatenate([bn_gamma[0:10], z6, z16]),
      jnp.concatenate([bn_beta[0:10], z6, z16]),
      jnp.concatenate([bn_gamma[10:20], z6, z16]),
      jnp.concatenate([bn_beta[10:20], z6, z16]),
      jnp.concatenate([spread10(bn_gamma[20:26]), z6, z16]),
      jnp.concatenate([spread10(bn_beta[20:26]), z6, z16]),
      b1,
      jnp.concatenate([b2, z16]),
      jnp.concatenate([b3, jnp.zeros((28,), jnp.float32)]),
  ] + [jnp.zeros((32,), jnp.float32)] * 7, axis=0)

  zc = jnp.zeros((32, 6), jnp.float32)
  z2c = jnp.zeros((32, 2), jnp.float32)
  wall = jnp.concatenate([
      W1[:, 0:10], zc,
      W1[:, 10:20], zc,
      W1[:, 20:21], z2c, W1[:, 21:26], z2c, zc,
      jnp.concatenate([W2, prow], axis=0),
      jnp.pad(W3, ((0, 28), (0, 0))),
  ], axis=1)
  return wall


def kernel(x, country_table, occupation_table, bn_gamma, bn_beta,
           W1, b1, W2, b2, W3, b3):
  xt_all = x.T
  iall = jnp.stack([xt_all[1], xt_all[8]]).astype(jnp.int32)
  tabs = jnp.concatenate([
      _flat_table(country_table), _flat_table(occupation_table),
      lax.bitcast_convert_type(iall, jnp.float32).reshape(2 * B)])
  co = _sc_gather(tabs)
  wall = _pack_params(bn_gamma, bn_beta, W1, b1, W2, b2, W3, b3)
  y3 = _tc_dense(co, xt_all, wall)
  return y3.T


# speedup vs baseline: 1.0340x; 1.0340x over previous
import functools

import jax
import jax.numpy as jnp
from jax import lax
from jax.experimental import pallas as pl
from jax.experimental.pallas import tpu as pltpu
from jax.experimental.pallas import tpu_sc as plsc

B = 16384
NC, NS = 2, 16
NW = NC * NS
BPT = B // (NW // 2)
GSZ = 16
TFLAT = 10 * 1024
EPS = 1e-5


def _sc_gather_body(tabs_hbm, out_hbm, tv, iv, ov, sem):
  wid = lax.axis_index("s") * NC + lax.axis_index("c")
  half = wid // (NW // 2)
  base = (wid % (NW // 2)) * BPT

  c1 = pltpu.async_copy(tabs_hbm.at[pl.ds(half * TFLAT, TFLAT)], tv, sem)
  c2 = pltpu.async_copy(
      tabs_hbm.at[pl.ds(2 * TFLAT + half * B + base, BPT)], iv, sem)
  c1.wait()
  c2.wait()

  @plsc.parallel_loop(0, BPT // GSZ, unroll=8)
  def _group(g):
    sl = pl.ds(g * GSZ, GSZ)
    pc = plsc.bitcast(iv[sl], jnp.int32)
    for d in range(10):
      ov[d, sl] = plsc.load_gather(tv, [pc + d * 1024])

  zeros = jnp.zeros((GSZ,), jnp.float32)
  for k in range(BPT // GSZ):
    sl = pl.ds(k * GSZ, GSZ)
    for d in range(10, 16):
      ov[d, sl] = zeros

  pltpu.sync_copy(ov, out_hbm.at[pl.ds(half * 16, 16), pl.ds(base, BPT)])


def _sc_gather(tabs):
  mesh = plsc.VectorSubcoreMesh(core_axis_name="c", subcore_axis_name="s")
  fn = functools.partial(
      pl.kernel,
      out_type=jax.ShapeDtypeStruct((32, B), jnp.float32),
      mesh=mesh,
      scratch_types=[
          pltpu.VMEM((TFLAT,), jnp.float32),
          pltpu.VMEM((BPT,), jnp.float32),
          pltpu.VMEM((16, BPT), jnp.float32),
          pltpu.SemaphoreType.DMA,
      ],
      compiler_params=pltpu.CompilerParams(needs_layout_passes=False),
  )(_sc_gather_body)
  return fn(tabs)


def _tc_dense_body(co_ref, xt_ref, wall_ref, out_ref):
  inv_b = 1.0 / B

  def bn(xp, g, b):
    m = jnp.sum(xp, axis=1, keepdims=True) * inv_b
    d = xp - m
    v = jnp.sum(d * d, axis=1, keepdims=True) * inv_b
    s = g * lax.rsqrt(v + EPS)
    return d * s + b

  pc = jnp.transpose(wall_ref[16:32, 48:80])
  wall = wall_ref[...]
  co = co_ref[...]
  xt = xt_ref[...]
  xc = bn(co[0:16], pc[0:16, 0:1], pc[0:16, 1:2])
  xo = bn(co[16:32], pc[0:16, 2:3], pc[0:16, 3:4])
  xn = bn(xt, pc[0:10, 4:5], pc[0:10, 5:6])
  y1 = (jnp.dot(wall[:, 0:16], xc, preferred_element_type=jnp.float32)
        + jnp.dot(wall[:, 16:32], xo, preferred_element_type=jnp.float32)
        + jnp.dot(wall[:, 32:42], xn, preferred_element_type=jnp.float32)
        + pc[:, 6:7])
  h1 = jnp.maximum(y1, 0.0)
  h2 = jnp.maximum(
      jnp.dot(wall[0:16, 48:80], h1, preferred_element_type=jnp.float32)
      + pc[0:16, 7:8], 0.0)
  out_ref[...] = (jnp.dot(wall[0:4, 80:96], h2,
                          preferred_element_type=jnp.float32)
                  + pc[0:4, 8:9])


def _tc_dense(co, xt, wall, interpret=False):
  return pl.pallas_call(
      _tc_dense_body,
      out_shape=jax.ShapeDtypeStruct((4, B), jnp.float32),
      interpret=interpret,
  )(co, xt, wall)


def _flat_table(tab):
  return jnp.pad(tab.T, ((0, 0), (0, 24))).reshape(TFLAT)


def _pack_params(bn_gamma, bn_beta, W1, b1, W2, b2, W3, b3):
  z6 = jnp.zeros((6,), jnp.float32)
  z16 = jnp.zeros((16,), jnp.float32)
  z2 = jnp.zeros((2,), jnp.float32)

  def spread10(v6):
    return jnp.concatenate([v6[0:1], z2, v6[1:6], z2])

  prow = jnp.stack([
      jnp.concatenate([bn_gamma[0:10], z6, z16]),
      jnp.concatenate([bn_beta[0:10], z6, z16]),
      jnp.concatenate([bn_gamma[10:20], z6, z16]),
      jnp.concatenate([bn_beta[10:20], z6, z16]),
      jnp.concatenate([spread10(bn_gamma[20:26]), z6, z16]),
      jnp.concatenate([spread10(bn_beta[20:26]), z6, z16]),
      b1,
      jnp.concatenate([b2, z16]),
      jnp.concatenate([b3, jnp.zeros((28,), jnp.float32)]),
  ] + [jnp.zeros((32,), jnp.float32)] * 7, axis=0)

  zc = jnp.zeros((32, 6), jnp.float32)
  z2c = jnp.zeros((32, 2), jnp.float32)
  wall = jnp.concatenate([
      W1[:, 0:10], zc,
      W1[:, 10:20], zc,
      W1[:, 20:21], z2c, W1[:, 21:26], z2c, zc,
      jnp.concatenate([W2, prow], axis=0),
      jnp.pad(W3, ((0, 28), (0, 0))),
  ], axis=1)
  return wall


def kernel(x, country_table, occupation_table, bn_gamma, bn_beta,
           W1, b1, W2, b2, W3, b3):
  xt_all = x.T
  iall = jnp.stack([xt_all[1], xt_all[8]]).astype(jnp.int32)
  tabs = jnp.concatenate([
      _flat_table(country_table), _flat_table(occupation_table),
      lax.bitcast_convert_type(iall, jnp.float32).reshape(2 * B)])
  co = _sc_gather(tabs)
  wall = _pack_params(bn_gamma, bn_beta, W1, b1, W2, b2, W3, b3)
  y3 = _tc_dense(co, xt_all, wall)
  return y3.T
